# baseline (speedup 1.0000x reference)
import functools

import jax
import jax.numpy as jnp
from jax import lax
from jax.experimental import pallas as pl
from jax.experimental.pallas import tpu as pltpu
from jax.experimental.pallas import tpu_sc as plsc

N = 10000
E = 320000
D = 128
NC, NS, L = 2, 16, 16
NW = NC * NS
CHUNK = 128
C = 80
EPT = C * CHUNK
E_PAD = NW * EPT
G = 16
GROUPS = C // G
T_CHUNKS = NW * C
CA = 160
CB = 2 * C - CA
N_PAD = 10240
NB = N_PAD // D
RPS = N_PAD // NS
RB = 1024


def _sc_mesh():
    return plsc.VectorSubcoreMesh(
        core_axis_name="c", subcore_axis_name="s", num_cores=NC, num_subcores=NS
    )


def _deg_body(dst_hbm, ew_hbm, degp_hbm, dstv, ewv, degs, zv):
    c = lax.axis_index("c")
    s = lax.axis_index("s")
    wid = c * NS + s
    zero = jnp.zeros((L,), jnp.float32)
    for i in range(RPS // L):
        zv[pl.ds(i * L, L)] = zero
    pltpu.sync_copy(zv, degs.at[pl.ds(s * RPS, RPS)])
    plsc.subcore_barrier()
    pltpu.sync_copy(dst_hbm.at[pl.ds(wid * C, C)], dstv)
    pltpu.sync_copy(ew_hbm.at[pl.ds(wid * C, C)], ewv)

    def body(j, carry):
        pltpu.sync_copy(ewv.at[j], degs.at[dstv.at[j]], add=True)
        return carry

    lax.fori_loop(0, C, body, 0)
    plsc.subcore_barrier()
    pltpu.sync_copy(degs.at[pl.ds(s * RPS, RPS)], degp_hbm.at[c, pl.ds(s * RPS, RPS)])


def _deg_partials(dst3, ew3):
    return pl.kernel(
        _deg_body,
        out_type=jax.ShapeDtypeStruct((NC, N_PAD), jnp.float32),
        mesh=_sc_mesh(),
        scratch_types=[
            pltpu.VMEM((C, CHUNK), jnp.int32),
            pltpu.VMEM((C, CHUNK), jnp.float32),
            pltpu.VMEM_SHARED((N_PAD,), jnp.float32),
            pltpu.VMEM((RPS,), jnp.float32),
        ],
        compiler_params=pltpu.CompilerParams(needs_layout_passes=False),
    )(dst3, ew3)


def _mp_body(hws_hbm, src_hbm, dst_hbm, ew_hbm, part_hbm,
             srcv, dstv, ewv, rows, acc, g0, g1, s0, s1, isem):
    c = lax.axis_index("c")
    s_ = lax.axis_index("s")
    wid = c * NS + s_
    zero = jnp.zeros((L,), jnp.float32)

    def zrow(i, carry):
        for k in range(D // L):
            rows[0, i, pl.ds(k * L, L)] = zero
        return carry

    lax.fori_loop(0, CHUNK, zrow, 0)
    for m in range(RPS // CHUNK):
        pltpu.sync_copy(rows.at[0], acc.at[pl.ds(s_ * RPS + m * CHUNK, CHUNK)])
    plsc.subcore_barrier()

    cbase = jnp.where(c == 0, s_ * CA, NS * CA + s_ * CB)
    ngroups = jnp.where(c == 0, CA // G, CB // G)

    def stage(g, par):
        b0 = cbase + g * G
        pltpu.async_copy(src_hbm.at[pl.ds(b0, G)], srcv.at[par], isem)
        pltpu.async_copy(dst_hbm.at[pl.ds(b0, G)], dstv.at[par], isem)
        pltpu.async_copy(ew_hbm.at[pl.ds(b0, G)], ewv.at[par], isem)

    def stage_wait(g, par):
        b0 = cbase + g * G
        pltpu.make_async_copy(src_hbm.at[pl.ds(b0, G)], srcv.at[par], isem).wait()
        pltpu.make_async_copy(dst_hbm.at[pl.ds(b0, G)], dstv.at[par], isem).wait()
        pltpu.make_async_copy(ew_hbm.at[pl.ds(b0, G)], ewv.at[par], isem).wait()

    @pl.when(ngroups > 0)
    def _():
        stage(0, 0)

    def group(g, carry):
        par = lax.rem(g, 2)
        stage_wait(g, par)

        @pl.when(g + 1 < ngroups)
        def _():
            stage(g + 1, 1 - par)

        def start_gather(j, b, sem):
            pltpu.async_copy(hws_hbm.at[srcv.at[par, j]], rows.at[b], sem)

        def wait_gather(j, b, sem):
            pltpu.make_async_copy(hws_hbm.at[srcv.at[par, j]], rows.at[b], sem).wait()

        def start_scatter(j, b, sem):
            pltpu.async_copy(rows.at[b], acc.at[dstv.at[par, j]], sem, add=True)

        def wait_scatter(j, b, sem):
            pltpu.make_async_copy(rows.at[b], acc.at[dstv.at[par, j]], sem).wait()

        def scale(j, b):
            @plsc.parallel_loop(0, CHUNK, unroll=4)
            def _(e):
                nv = plsc.load_gather(
                    ewv,
                    [jnp.full((L,), par, jnp.int32), jnp.full((L,), j, jnp.int32),
                     jnp.full((L,), e, jnp.int32)],
                )
                for k in range(D // L):
                    sl = pl.ds(k * L, L)
                    rows[b, e, sl] = rows[b, e, sl] * nv

        start_gather(0, 0, g0)
        wait_gather(0, 0, g0)
        start_gather(1, 1, g1)
        scale(0, 0)
        start_scatter(0, 0, s0)
        wait_gather(1, 1, g1)
        wait_scatter(0, 0, s0)
        start_gather(2, 0, g0)
        scale(1, 1)
        start_scatter(1, 1, s1)

        def pair(p, cc):
            j0 = 2 * p
            wait_gather(j0, 0, g0)
            wait_scatter(j0 - 1, 1, s1)
            start_gather(j0 + 1, 1, g1)
            scale(j0, 0)
            start_scatter(j0, 0, s0)
            wait_gather(j0 + 1, 1, g1)
            wait_scatter(j0, 0, s0)
            start_gather(j0 + 2, 0, g0)
            scale(j0 + 1, 1)
            start_scatter(j0 + 1, 1, s1)
            return cc

        lax.fori_loop(1, G // 2 - 1, pair, 0)
        jl = G - 2
        wait_gather(jl, 0, g0)
        wait_scatter(jl - 1, 1, s1)
        start_gather(jl + 1, 1, g1)
        scale(jl, 0)
        start_scatter(jl, 0, s0)
        wait_gather(jl + 1, 1, g1)
        scale(jl + 1, 1)
        start_scatter(jl + 1, 1, s1)
        wait_scatter(jl, 0, s0)
        wait_scatter(jl + 1, 1, s1)
        return carry

    lax.fori_loop(0, ngroups, group, 0)
    plsc.subcore_barrier()
    for m in range(RPS // CHUNK):
        r0 = s_ * RPS + m * CHUNK
        pltpu.sync_copy(acc.at[pl.ds(r0, CHUNK)], part_hbm.at[c, pl.ds(r0, CHUNK)])


def _message_pass(hws, src3, dst3, ew3):
    return pl.kernel(
        _mp_body,
        out_type=jax.ShapeDtypeStruct((NC, N_PAD, D), jnp.float32),
        mesh=_sc_mesh(),
        scratch_types=[
            pltpu.VMEM((2, G, CHUNK), jnp.int32),
            pltpu.VMEM((2, G, CHUNK), jnp.int32),
            pltpu.VMEM((2, G, CHUNK), jnp.float32),
            pltpu.VMEM((2, CHUNK, D), jnp.float32),
            pltpu.VMEM_SHARED((N_PAD, D), jnp.float32),
            pltpu.SemaphoreType.DMA,
            pltpu.SemaphoreType.DMA,
            pltpu.SemaphoreType.DMA,
            pltpu.SemaphoreType.DMA,
            pltpu.SemaphoreType.DMA,
        ],
        compiler_params=pltpu.CompilerParams(needs_layout_passes=False),
    )(hws, src3, dst3, ew3)


def _fin_body(degp_ref, dis_ref, dinv_ref):
    deg = degp_ref[0] + degp_ref[1] + 1.0
    dis_ref[...] = lax.rsqrt(deg)
    dinv_ref[...] = 1.0 / deg


def _finalize(degp):
    return pl.pallas_call(
        _fin_body,
        out_shape=[jax.ShapeDtypeStruct((NB, D), jnp.float32)] * 2,
    )(degp.reshape(NC, NB, D))


def _mm_body(h_ref, w_ref, disc_ref, hw_ref, hws_ref):
    hw = jnp.dot(h_ref[...], w_ref[...], preferred_element_type=jnp.float32)
    hw_ref[...] = hw
    hws_ref[...] = hw * disc_ref[...]


def _matmul_scaled(h, W, disc):
    grid = (N_PAD // RB,)
    return pl.pallas_call(
        _mm_body,
        grid=grid,
        in_specs=[
            pl.BlockSpec((RB, D), lambda i: (i, 0)),
            pl.BlockSpec((D, D), lambda i: (0, 0)),
            pl.BlockSpec((RB, 1), lambda i: (i, 0)),
        ],
        out_specs=[pl.BlockSpec((RB, D), lambda i: (i, 0))] * 2,
        out_shape=[jax.ShapeDtypeStruct((N_PAD, D), jnp.float32)] * 2,
    )(h, W, disc)


def _epmm_body(part_ref, hw_ref, h_ref, disc_ref, dinvc_ref, b_ref, g_ref,
               be_ref, w_ref, hn_ref, hw2_ref, hws2_ref):
    o = (part_ref[0] + part_ref[1]) * disc_ref[...]
    o = o + hw_ref[...] * dinvc_ref[...] + b_ref[0][None, :]
    mu = jnp.mean(o, axis=-1, keepdims=True)
    v = jnp.mean((o - mu) ** 2, axis=-1, keepdims=True)
    o = (o - mu) * lax.rsqrt(v + 1e-5) * g_ref[0][None, :] + be_ref[0][None, :]
    hn = jnp.maximum(o, 0.0) + h_ref[...]
    hn_ref[...] = hn
    hw2 = jnp.dot(hn, w_ref[...], preferred_element_type=jnp.float32)
    hw2_ref[...] = hw2
    hws2_ref[...] = hw2 * disc_ref[...]


def _ep_matmul(part, hw, h, disc, dinvc, b, g, be, Wn):
    grid = (N_PAD // RB,)
    return pl.pallas_call(
        _epmm_body,
        grid=grid,
        in_specs=[
            pl.BlockSpec((NC, RB, D), lambda i: (0, i, 0)),
            pl.BlockSpec((RB, D), lambda i: (i, 0)),
            pl.BlockSpec((RB, D), lambda i: (i, 0)),
            pl.BlockSpec((RB, 1), lambda i: (i, 0)),
            pl.BlockSpec((RB, 1), lambda i: (i, 0)),
            pl.BlockSpec((1, D), lambda i: (0, 0)),
            pl.BlockSpec((1, D), lambda i: (0, 0)),
            pl.BlockSpec((1, D), lambda i: (0, 0)),
            pl.BlockSpec((D, D), lambda i: (0, 0)),
        ],
        out_specs=[pl.BlockSpec((RB, D), lambda i: (i, 0))] * 3,
        out_shape=[jax.ShapeDtypeStruct((N_PAD, D), jnp.float32)] * 3,
    )(part, hw, h, disc, dinvc, b, g, be, Wn)


def _ep_body(part_ref, hw_ref, h_ref, disc_ref, dinvc_ref, b_ref, g_ref, be_ref, out_ref, *, relu):
    o = (part_ref[0] + part_ref[1]) * disc_ref[...]
    o = o + hw_ref[...] * dinvc_ref[...] + b_ref[0][None, :]
    mu = jnp.mean(o, axis=-1, keepdims=True)
    v = jnp.mean((o - mu) ** 2, axis=-1, keepdims=True)
    o = (o - mu) * lax.rsqrt(v + 1e-5) * g_ref[0][None, :] + be_ref[0][None, :]
    if relu:
        o = jnp.maximum(o, 0.0)
    out_ref[...] = o + h_ref[...]


def _epilogue(part, hw, h, disc, dinvc, b, g, be, relu):
    grid = (N_PAD // RB,)
    return pl.pallas_call(
        functools.partial(_ep_body, relu=relu),
        grid=grid,
        in_specs=[
            pl.BlockSpec((NC, RB, D), lambda i: (0, i, 0)),
            pl.BlockSpec((RB, D), lambda i: (i, 0)),
            pl.BlockSpec((RB, D), lambda i: (i, 0)),
            pl.BlockSpec((RB, 1), lambda i: (i, 0)),
            pl.BlockSpec((RB, 1), lambda i: (i, 0)),
            pl.BlockSpec((1, D), lambda i: (0, 0)),
            pl.BlockSpec((1, D), lambda i: (0, 0)),
            pl.BlockSpec((1, D), lambda i: (0, 0)),
        ],
        out_specs=pl.BlockSpec((RB, D), lambda i: (i, 0)),
        out_shape=jax.ShapeDtypeStruct((N_PAD, D), jnp.float32),
    )(part, hw, h, disc, dinvc, b, g, be)


def kernel(x, edge_index, edge_weight, W1, b1, W2, b2, W3, b3, g1, be1, g2, be2, g3, be3):
    src = edge_index[0].astype(jnp.int32)
    dst = edge_index[1].astype(jnp.int32)
    ew = edge_weight.astype(jnp.float32)
    pad = E_PAD - E
    src3 = jnp.pad(src, (0, pad)).reshape(T_CHUNKS, CHUNK)
    dst3 = jnp.pad(dst, (0, pad)).reshape(T_CHUNKS, CHUNK)
    ew3 = jnp.pad(ew, (0, pad)).reshape(T_CHUNKS, CHUNK)

    degp = _deg_partials(dst3, ew3)
    dis2, dinv2 = _finalize(degp)
    disc = dis2.reshape(N_PAD, 1)
    dinvc = dinv2.reshape(N_PAD, 1)

    h = jnp.pad(x, ((0, N_PAD - N), (0, 0)))
    hw, hws = _matmul_scaled(h, W1, disc)
    part = _message_pass(hws, src3, dst3, ew3)
    h1, hw2, hws2 = _ep_matmul(part, hw, h, disc, dinvc,
                               b1.reshape(1, D), g1.reshape(1, D),
                               be1.reshape(1, D), W2)
    part = _message_pass(hws2, src3, dst3, ew3)
    h2, hw3, hws3 = _ep_matmul(part, hw2, h1, disc, dinvc,
                               b2.reshape(1, D), g2.reshape(1, D),
                               be2.reshape(1, D), W3)
    part = _message_pass(hws3, src3, dst3, ew3)
    h3 = _epilogue(part, hw3, h2, disc, dinvc,
                   b3.reshape(1, D), g3.reshape(1, D), be3.reshape(1, D), False)
    return h3[:N]


# speedup vs baseline: 1.5452x; 1.5452x over previous
import functools

import jax
import jax.numpy as jnp
from jax import lax
from jax.experimental import pallas as pl
from jax.experimental.pallas import tpu as pltpu
from jax.experimental.pallas import tpu_sc as plsc

N = 10000
E = 320000
D = 128
NC, NS, L = 2, 16, 16
NW = NC * NS
CHUNK = 128
C = 80
EPT = C * CHUNK
E_PAD = NW * EPT
G = 16
GROUPS = C // G
T_CHUNKS = NW * C
CA = 144
CB = 2 * C - CA
N_PAD = 10240
NB = N_PAD // D
RPS = N_PAD // NS
RB = 1024


def _sc_mesh():
    return plsc.VectorSubcoreMesh(
        core_axis_name="c", subcore_axis_name="s", num_cores=NC, num_subcores=NS
    )


def _deg_body(dst_hbm, ew_hbm, degp_hbm, dstv, ewv, degs, zv):
    c = lax.axis_index("c")
    s = lax.axis_index("s")
    wid = c * NS + s
    zero = jnp.zeros((L,), jnp.float32)
    for i in range(RPS // L):
        zv[pl.ds(i * L, L)] = zero
    pltpu.sync_copy(zv, degs.at[pl.ds(s * RPS, RPS)])
    plsc.subcore_barrier()
    pltpu.sync_copy(dst_hbm.at[pl.ds(wid * C, C)], dstv)
    pltpu.sync_copy(ew_hbm.at[pl.ds(wid * C, C)], ewv)

    def body(j, carry):
        pltpu.sync_copy(ewv.at[j], degs.at[dstv.at[j]], add=True)
        return carry

    lax.fori_loop(0, C, body, 0)
    plsc.subcore_barrier()
    pltpu.sync_copy(degs.at[pl.ds(s * RPS, RPS)], degp_hbm.at[c, pl.ds(s * RPS, RPS)])


def _deg_partials(dst3, ew3):
    return pl.kernel(
        _deg_body,
        out_type=jax.ShapeDtypeStruct((NC, N_PAD), jnp.float32),
        mesh=_sc_mesh(),
        scratch_types=[
            pltpu.VMEM((C, CHUNK), jnp.int32),
            pltpu.VMEM((C, CHUNK), jnp.float32),
            pltpu.VMEM_SHARED((N_PAD,), jnp.float32),
            pltpu.VMEM((RPS,), jnp.float32),
        ],
        compiler_params=pltpu.CompilerParams(needs_layout_passes=False),
    )(dst3, ew3)


def _mp_body(hws_hbm, src_hbm, dst_hbm, ew_hbm, part_hbm,
             srcv, dstv, ewv, rows, acc, g0, g1, s0, s1, isem):
    c = lax.axis_index("c")
    s_ = lax.axis_index("s")
    wid = c * NS + s_
    zero = jnp.zeros((L,), jnp.float32)

    def zrow(i, carry):
        for k in range(D // L):
            rows[0, i, pl.ds(k * L, L)] = zero
        return carry

    lax.fori_loop(0, CHUNK, zrow, 0)
    for m in range(RPS // CHUNK):
        pltpu.sync_copy(rows.at[0], acc.at[pl.ds(s_ * RPS + m * CHUNK, CHUNK)])
    plsc.subcore_barrier()

    cbase = jnp.where(c == 0, s_ * CA, NS * CA + s_ * CB)
    ngroups = jnp.where(c == 0, CA // G, CB // G)

    def stage(g, par):
        b0 = cbase + g * G
        pltpu.async_copy(src_hbm.at[pl.ds(b0, G)], srcv.at[par], isem)
        pltpu.async_copy(dst_hbm.at[pl.ds(b0, G)], dstv.at[par], isem)
        pltpu.async_copy(ew_hbm.at[pl.ds(b0, G)], ewv.at[par], isem)

    def stage_wait(g, par):
        b0 = cbase + g * G
        pltpu.make_async_copy(src_hbm.at[pl.ds(b0, G)], srcv.at[par], isem).wait()
        pltpu.make_async_copy(dst_hbm.at[pl.ds(b0, G)], dstv.at[par], isem).wait()
        pltpu.make_async_copy(ew_hbm.at[pl.ds(b0, G)], ewv.at[par], isem).wait()

    @pl.when(ngroups > 0)
    def _():
        stage(0, 0)

    def group(g, carry):
        par = lax.rem(g, 2)
        stage_wait(g, par)

        @pl.when(g + 1 < ngroups)
        def _():
            stage(g + 1, 1 - par)

        def start_gather(j, b, sem):
            pltpu.async_copy(hws_hbm.at[srcv.at[par, j]], rows.at[b], sem)

        def wait_gather(j, b, sem):
            pltpu.make_async_copy(hws_hbm.at[srcv.at[par, j]], rows.at[b], sem).wait()

        def start_scatter(j, b, sem):
            pltpu.async_copy(rows.at[b], acc.at[dstv.at[par, j]], sem, add=True)

        def wait_scatter(j, b, sem):
            pltpu.make_async_copy(rows.at[b], acc.at[dstv.at[par, j]], sem).wait()

        def scale(j, b):
            @plsc.parallel_loop(0, CHUNK, unroll=8)
            def _(e):
                nv = plsc.load_gather(
                    ewv,
                    [jnp.full((L,), par, jnp.int32), jnp.full((L,), j, jnp.int32),
                     jnp.full((L,), e, jnp.int32)],
                )
                for k in range(D // L):
                    sl = pl.ds(k * L, L)
                    rows[b, e, sl] = rows[b, e, sl] * nv

        start_gather(0, 0, g0)
        wait_gather(0, 0, g0)
        start_gather(1, 1, g1)
        scale(0, 0)
        start_scatter(0, 0, s0)
        wait_gather(1, 1, g1)
        wait_scatter(0, 0, s0)
        start_gather(2, 0, g0)
        scale(1, 1)
        start_scatter(1, 1, s1)

        def pair(p, cc):
            j0 = 2 * p
            wait_gather(j0, 0, g0)
            wait_scatter(j0 - 1, 1, s1)
            start_gather(j0 + 1, 1, g1)
            scale(j0, 0)
            start_scatter(j0, 0, s0)
            wait_gather(j0 + 1, 1, g1)
            wait_scatter(j0, 0, s0)
            start_gather(j0 + 2, 0, g0)
            scale(j0 + 1, 1)
            start_scatter(j0 + 1, 1, s1)
            return cc

        lax.fori_loop(1, G // 2 - 1, pair, 0)
        jl = G - 2
        wait_gather(jl, 0, g0)
        wait_scatter(jl - 1, 1, s1)
        start_gather(jl + 1, 1, g1)
        scale(jl, 0)
        start_scatter(jl, 0, s0)
        wait_gather(jl + 1, 1, g1)
        scale(jl + 1, 1)
        start_scatter(jl + 1, 1, s1)
        wait_scatter(jl, 0, s0)
        wait_scatter(jl + 1, 1, s1)
        return carry

    lax.fori_loop(0, ngroups, group, 0)
    plsc.subcore_barrier()
    for m in range(RPS // CHUNK):
        r0 = s_ * RPS + m * CHUNK
        pltpu.sync_copy(acc.at[pl.ds(r0, CHUNK)], part_hbm.at[c, pl.ds(r0, CHUNK)])


def _message_pass(hws, src3, dst3, ew3):
    return pl.kernel(
        _mp_body,
        out_type=jax.ShapeDtypeStruct((NC, N_PAD, D), jnp.float32),
        mesh=_sc_mesh(),
        scratch_types=[
            pltpu.VMEM((2, G, CHUNK), jnp.int32),
            pltpu.VMEM((2, G, CHUNK), jnp.int32),
            pltpu.VMEM((2, G, CHUNK), jnp.float32),
            pltpu.VMEM((2, CHUNK, D), jnp.float32),
            pltpu.VMEM_SHARED((N_PAD, D), jnp.float32),
            pltpu.SemaphoreType.DMA,
            pltpu.SemaphoreType.DMA,
            pltpu.SemaphoreType.DMA,
            pltpu.SemaphoreType.DMA,
            pltpu.SemaphoreType.DMA,
        ],
        compiler_params=pltpu.CompilerParams(needs_layout_passes=False),
    )(hws, src3, dst3, ew3)


def _fin_body(degp_ref, dis_ref, dinv_ref):
    deg = degp_ref[0] + degp_ref[1] + 1.0
    dis_ref[...] = lax.rsqrt(deg)
    dinv_ref[...] = 1.0 / deg


def _finalize(degp):
    return pl.pallas_call(
        _fin_body,
        out_shape=[jax.ShapeDtypeStruct((NB, D), jnp.float32)] * 2,
    )(degp.reshape(NC, NB, D))


def _mm_body(h_ref, w_ref, disc_ref, hw_ref, hws_ref):
    hw = jnp.dot(h_ref[...], w_ref[...], preferred_element_type=jnp.float32)
    hw_ref[...] = hw
    hws_ref[...] = hw * disc_ref[...]


def _matmul_scaled(h, W, disc):
    grid = (N_PAD // RB,)
    return pl.pallas_call(
        _mm_body,
        grid=grid,
        in_specs=[
            pl.BlockSpec((RB, D), lambda i: (i, 0)),
            pl.BlockSpec((D, D), lambda i: (0, 0)),
            pl.BlockSpec((RB, 1), lambda i: (i, 0)),
        ],
        out_specs=[pl.BlockSpec((RB, D), lambda i: (i, 0))] * 2,
        out_shape=[jax.ShapeDtypeStruct((N_PAD, D), jnp.float32)] * 2,
    )(h, W, disc)


def _epmm_body(part_ref, hw_ref, h_ref, disc_ref, dinvc_ref, b_ref, g_ref,
               be_ref, w_ref, hn_ref, hw2_ref, hws2_ref):
    o = (part_ref[0] + part_ref[1]) * disc_ref[...]
    o = o + hw_ref[...] * dinvc_ref[...] + b_ref[0][None, :]
    mu = jnp.mean(o, axis=-1, keepdims=True)
    v = jnp.mean((o - mu) ** 2, axis=-1, keepdims=True)
    o = (o - mu) * lax.rsqrt(v + 1e-5) * g_ref[0][None, :] + be_ref[0][None, :]
    hn = jnp.maximum(o, 0.0) + h_ref[...]
    hn_ref[...] = hn
    hw2 = jnp.dot(hn, w_ref[...], preferred_element_type=jnp.float32)
    hw2_ref[...] = hw2
    hws2_ref[...] = hw2 * disc_ref[...]


def _ep_matmul(part, hw, h, disc, dinvc, b, g, be, Wn):
    grid = (N_PAD // RB,)
    return pl.pallas_call(
        _epmm_body,
        grid=grid,
        in_specs=[
            pl.BlockSpec((NC, RB, D), lambda i: (0, i, 0)),
            pl.BlockSpec((RB, D), lambda i: (i, 0)),
            pl.BlockSpec((RB, D), lambda i: (i, 0)),
            pl.BlockSpec((RB, 1), lambda i: (i, 0)),
            pl.BlockSpec((RB, 1), lambda i: (i, 0)),
            pl.BlockSpec((1, D), lambda i: (0, 0)),
            pl.BlockSpec((1, D), lambda i: (0, 0)),
            pl.BlockSpec((1, D), lambda i: (0, 0)),
            pl.BlockSpec((D, D), lambda i: (0, 0)),
        ],
        out_specs=[pl.BlockSpec((RB, D), lambda i: (i, 0))] * 3,
        out_shape=[jax.ShapeDtypeStruct((N_PAD, D), jnp.float32)] * 3,
    )(part, hw, h, disc, dinvc, b, g, be, Wn)


def _ep_body(part_ref, hw_ref, h_ref, disc_ref, dinvc_ref, b_ref, g_ref, be_ref, out_ref, *, relu):
    o = (part_ref[0] + part_ref[1]) * disc_ref[...]
    o = o + hw_ref[...] * dinvc_ref[...] + b_ref[0][None, :]
    mu = jnp.mean(o, axis=-1, keepdims=True)
    v = jnp.mean((o - mu) ** 2, axis=-1, keepdims=True)
    o = (o - mu) * lax.rsqrt(v + 1e-5) * g_ref[0][None, :] + be_ref[0][None, :]
    if relu:
        o = jnp.maximum(o, 0.0)
    out_ref[...] = o + h_ref[...]


def _epilogue(part, hw, h, disc, dinvc, b, g, be, relu):
    grid = (N_PAD // RB,)
    return pl.pallas_call(
        functools.partial(_ep_body, relu=relu),
        grid=grid,
        in_specs=[
            pl.BlockSpec((NC, RB, D), lambda i: (0, i, 0)),
            pl.BlockSpec((RB, D), lambda i: (i, 0)),
            pl.BlockSpec((RB, D), lambda i: (i, 0)),
            pl.BlockSpec((RB, 1), lambda i: (i, 0)),
            pl.BlockSpec((RB, 1), lambda i: (i, 0)),
            pl.BlockSpec((1, D), lambda i: (0, 0)),
            pl.BlockSpec((1, D), lambda i: (0, 0)),
            pl.BlockSpec((1, D), lambda i: (0, 0)),
        ],
        out_specs=pl.BlockSpec((RB, D), lambda i: (i, 0)),
        out_shape=jax.ShapeDtypeStruct((N_PAD, D), jnp.float32),
    )(part, hw, h, disc, dinvc, b, g, be)


def kernel(x, edge_index, edge_weight, W1, b1, W2, b2, W3, b3, g1, be1, g2, be2, g3, be3):
    src = edge_index[0].astype(jnp.int32)
    dst = edge_index[1].astype(jnp.int32)
    ew = edge_weight.astype(jnp.float32)
    pad = E_PAD - E
    src3 = jnp.pad(src, (0, pad)).reshape(T_CHUNKS, CHUNK)
    dst3 = jnp.pad(dst, (0, pad)).reshape(T_CHUNKS, CHUNK)
    ew3 = jnp.pad(ew, (0, pad)).reshape(T_CHUNKS, CHUNK)

    degp = _deg_partials(dst3, ew3)
    dis2, dinv2 = _finalize(degp)
    disc = dis2.reshape(N_PAD, 1)
    dinvc = dinv2.reshape(N_PAD, 1)

    h = jnp.pad(x, ((0, N_PAD - N), (0, 0)))
    hw, hws = _matmul_scaled(h, W1, disc)
    part = _message_pass(hws, src3, dst3, ew3)
    h1, hw2, hws2 = _ep_matmul(part, hw, h, disc, dinvc,
                               b1.reshape(1, D), g1.reshape(1, D),
                               be1.reshape(1, D), W2)
    part = _message_pass(hws2, src3, dst3, ew3)
    h2, hw3, hws3 = _ep_matmul(part, hw2, h1, disc, dinvc,
                               b2.reshape(1, D), g2.reshape(1, D),
                               be2.reshape(1, D), W3)
    part = _message_pass(hws3, src3, dst3, ew3)
    h3 = _epilogue(part, hw3, h2, disc, dinvc,
                   b3.reshape(1, D), g3.reshape(1, D), be3.reshape(1, D), False)
    return h3[:N]


# speedup vs baseline: 1.5495x; 1.0028x over previous
import functools

import jax
import jax.numpy as jnp
from jax import lax
from jax.experimental import pallas as pl
from jax.experimental.pallas import tpu as pltpu
from jax.experimental.pallas import tpu_sc as plsc

N = 10000
E = 320000
D = 128
NC, NS, L = 2, 16, 16
NW = NC * NS
CHUNK = 128
C = 80
EPT = C * CHUNK
E_PAD = NW * EPT
G = 16
GROUPS = C // G
T_CHUNKS = NW * C
CA = 144
CB = 2 * C - CA
N_PAD = 10240
NB = N_PAD // D
RPS = N_PAD // NS
RB = 1024


def _sc_mesh():
    return plsc.VectorSubcoreMesh(
        core_axis_name="c", subcore_axis_name="s", num_cores=NC, num_subcores=NS
    )


def _deg_body(dst_hbm, ew_hbm, degp_hbm, dstv, ewv, degs, zv):
    c = lax.axis_index("c")
    s = lax.axis_index("s")
    wid = c * NS + s
    zero = jnp.zeros((L,), jnp.float32)
    for i in range(RPS // L):
        zv[pl.ds(i * L, L)] = zero
    pltpu.sync_copy(zv, degs.at[pl.ds(s * RPS, RPS)])
    plsc.subcore_barrier()
    pltpu.sync_copy(dst_hbm.at[pl.ds(wid * C, C)], dstv)
    pltpu.sync_copy(ew_hbm.at[pl.ds(wid * C, C)], ewv)

    def body(j, carry):
        pltpu.sync_copy(ewv.at[j], degs.at[dstv.at[j]], add=True)
        return carry

    lax.fori_loop(0, C, body, 0)
    plsc.subcore_barrier()
    pltpu.sync_copy(degs.at[pl.ds(s * RPS, RPS)], degp_hbm.at[c, pl.ds(s * RPS, RPS)])


def _deg_partials(dst3, ew3):
    return pl.kernel(
        _deg_body,
        out_type=jax.ShapeDtypeStruct((NC, N_PAD), jnp.float32),
        mesh=_sc_mesh(),
        scratch_types=[
            pltpu.VMEM((C, CHUNK), jnp.int32),
            pltpu.VMEM((C, CHUNK), jnp.float32),
            pltpu.VMEM_SHARED((N_PAD,), jnp.float32),
            pltpu.VMEM((RPS,), jnp.float32),
        ],
        compiler_params=pltpu.CompilerParams(needs_layout_passes=False),
    )(dst3, ew3)


def _mp_body(hws_hbm, src_hbm, dst_hbm, ew_hbm, part_hbm,
             srcv, dstv, ewv, rows, acc, g0, g1, s0, s1, isem):
    c = lax.axis_index("c")
    s_ = lax.axis_index("s")
    wid = c * NS + s_
    zero = jnp.zeros((L,), jnp.float32)

    def zrow(i, carry):
        for k in range(D // L):
            rows[0, i, pl.ds(k * L, L)] = zero
        return carry

    lax.fori_loop(0, CHUNK, zrow, 0)
    for m in range(RPS // CHUNK):
        pltpu.sync_copy(rows.at[0], acc.at[pl.ds(s_ * RPS + m * CHUNK, CHUNK)])
    plsc.subcore_barrier()

    cbase = jnp.where(c == 0, s_ * CA, NS * CA + s_ * CB)
    ngroups = jnp.where(c == 0, CA // G, CB // G)

    def stage(g, par):
        b0 = cbase + g * G
        pltpu.async_copy(src_hbm.at[pl.ds(b0, G)], srcv.at[par], isem)
        pltpu.async_copy(dst_hbm.at[pl.ds(b0, G)], dstv.at[par], isem)
        pltpu.async_copy(ew_hbm.at[pl.ds(b0, G)], ewv.at[par], isem)

    def stage_wait(g, par):
        b0 = cbase + g * G
        pltpu.make_async_copy(src_hbm.at[pl.ds(b0, G)], srcv.at[par], isem).wait()
        pltpu.make_async_copy(dst_hbm.at[pl.ds(b0, G)], dstv.at[par], isem).wait()
        pltpu.make_async_copy(ew_hbm.at[pl.ds(b0, G)], ewv.at[par], isem).wait()

    @pl.when(ngroups > 0)
    def _():
        stage(0, 0)

    def group(g, carry):
        par = lax.rem(g, 2)
        stage_wait(g, par)

        @pl.when(g + 1 < ngroups)
        def _():
            stage(g + 1, 1 - par)

        def start_gather(j, b, sem):
            pltpu.async_copy(hws_hbm.at[srcv.at[par, j]], rows.at[b], sem)

        def wait_gather(j, b, sem):
            pltpu.make_async_copy(hws_hbm.at[srcv.at[par, j]], rows.at[b], sem).wait()

        def start_scatter(j, b, sem):
            pltpu.async_copy(rows.at[b], acc.at[dstv.at[par, j]], sem, add=True)

        def wait_scatter(j, b, sem):
            pltpu.make_async_copy(rows.at[b], acc.at[dstv.at[par, j]], sem).wait()

        def scale(j, b):
            @plsc.parallel_loop(0, CHUNK, unroll=4)
            def _(e):
                nv = plsc.load_gather(
                    ewv,
                    [jnp.full((L,), par, jnp.int32), jnp.full((L,), j, jnp.int32),
                     jnp.full((L,), e, jnp.int32)],
                )
                for k in range(D // L):
                    sl = pl.ds(k * L, L)
                    rows[b, e, sl] = rows[b, e, sl] * nv

        start_gather(0, 0, g0)
        wait_gather(0, 0, g0)
        start_gather(1, 1, g1)
        scale(0, 0)
        start_scatter(0, 0, s0)
        wait_gather(1, 1, g1)
        wait_scatter(0, 0, s0)
        start_gather(2, 0, g0)
        scale(1, 1)
        start_scatter(1, 1, s1)

        def pair(p, cc):
            j0 = 2 * p
            wait_gather(j0, 0, g0)
            wait_scatter(j0 - 1, 1, s1)
            start_gather(j0 + 1, 1, g1)
            scale(j0, 0)
            start_scatter(j0, 0, s0)
            wait_gather(j0 + 1, 1, g1)
            wait_scatter(j0, 0, s0)
            start_gather(j0 + 2, 0, g0)
            scale(j0 + 1, 1)
            start_scatter(j0 + 1, 1, s1)
            return cc

        lax.fori_loop(1, G // 2 - 1, pair, 0)
        jl = G - 2
        wait_gather(jl, 0, g0)
        wait_scatter(jl - 1, 1, s1)
        start_gather(jl + 1, 1, g1)
        scale(jl, 0)
        start_scatter(jl, 0, s0)
        wait_gather(jl + 1, 1, g1)
        scale(jl + 1, 1)
        start_scatter(jl + 1, 1, s1)
        wait_scatter(jl, 0, s0)
        wait_scatter(jl + 1, 1, s1)
        return carry

    lax.fori_loop(0, ngroups, group, 0)
    plsc.subcore_barrier()
    for m in range(RPS // CHUNK):
        r0 = s_ * RPS + m * CHUNK
        pltpu.sync_copy(acc.at[pl.ds(r0, CHUNK)], part_hbm.at[c, pl.ds(r0, CHUNK)])


def _message_pass(hws, src3, dst3, ew3):
    return pl.kernel(
        _mp_body,
        out_type=jax.ShapeDtypeStruct((NC, N_PAD, D), jnp.float32),
        mesh=_sc_mesh(),
        scratch_types=[
            pltpu.VMEM((2, G, CHUNK), jnp.int32),
            pltpu.VMEM((2, G, CHUNK), jnp.int32),
            pltpu.VMEM((2, G, CHUNK), jnp.float32),
            pltpu.VMEM((2, CHUNK, D), jnp.float32),
            pltpu.VMEM_SHARED((N_PAD, D), jnp.float32),
            pltpu.SemaphoreType.DMA,
            pltpu.SemaphoreType.DMA,
            pltpu.SemaphoreType.DMA,
            pltpu.SemaphoreType.DMA,
            pltpu.SemaphoreType.DMA,
        ],
        compiler_params=pltpu.CompilerParams(needs_layout_passes=False),
    )(hws, src3, dst3, ew3)


def _fin_body(degp_ref, dis_ref, dinv_ref):
    deg = degp_ref[0] + degp_ref[1] + 1.0
    dis_ref[...] = lax.rsqrt(deg)
    dinv_ref[...] = 1.0 / deg


def _finalize(degp):
    return pl.pallas_call(
        _fin_body,
        out_shape=[jax.ShapeDtypeStruct((NB, D), jnp.float32)] * 2,
    )(degp.reshape(NC, NB, D))


def _mm_body(h_ref, w_ref, disc_ref, hw_ref, hws_ref):
    hw = jnp.dot(h_ref[...], w_ref[...], preferred_element_type=jnp.float32)
    hw_ref[...] = hw
    hws_ref[...] = hw * disc_ref[...]


def _matmul_scaled(h, W, disc):
    grid = (N_PAD // RB,)
    return pl.pallas_call(
        _mm_body,
        grid=grid,
        in_specs=[
            pl.BlockSpec((RB, D), lambda i: (i, 0)),
            pl.BlockSpec((D, D), lambda i: (0, 0)),
            pl.BlockSpec((RB, 1), lambda i: (i, 0)),
        ],
        out_specs=[pl.BlockSpec((RB, D), lambda i: (i, 0))] * 2,
        out_shape=[jax.ShapeDtypeStruct((N_PAD, D), jnp.float32)] * 2,
    )(h, W, disc)


def _epmm_body(part_ref, hw_ref, h_ref, disc_ref, dinvc_ref, b_ref, g_ref,
               be_ref, w_ref, hn_ref, hw2_ref, hws2_ref):
    o = (part_ref[0] + part_ref[1]) * disc_ref[...]
    o = o + hw_ref[...] * dinvc_ref[...] + b_ref[0][None, :]
    mu = jnp.mean(o, axis=-1, keepdims=True)
    v = jnp.mean((o - mu) ** 2, axis=-1, keepdims=True)
    o = (o - mu) * lax.rsqrt(v + 1e-5) * g_ref[0][None, :] + be_ref[0][None, :]
    hn = jnp.maximum(o, 0.0) + h_ref[...]
    hn_ref[...] = hn
    hw2 = jnp.dot(hn, w_ref[...], preferred_element_type=jnp.float32)
    hw2_ref[...] = hw2
    hws2_ref[...] = hw2 * disc_ref[...]


def _ep_matmul(part, hw, h, disc, dinvc, b, g, be, Wn):
    grid = (N_PAD // RB,)
    return pl.pallas_call(
        _epmm_body,
        grid=grid,
        in_specs=[
            pl.BlockSpec((NC, RB, D), lambda i: (0, i, 0)),
            pl.BlockSpec((RB, D), lambda i: (i, 0)),
            pl.BlockSpec((RB, D), lambda i: (i, 0)),
            pl.BlockSpec((RB, 1), lambda i: (i, 0)),
            pl.BlockSpec((RB, 1), lambda i: (i, 0)),
            pl.BlockSpec((1, D), lambda i: (0, 0)),
            pl.BlockSpec((1, D), lambda i: (0, 0)),
            pl.BlockSpec((1, D), lambda i: (0, 0)),
            pl.BlockSpec((D, D), lambda i: (0, 0)),
        ],
        out_specs=[pl.BlockSpec((RB, D), lambda i: (i, 0))] * 3,
        out_shape=[jax.ShapeDtypeStruct((N_PAD, D), jnp.float32)] * 3,
    )(part, hw, h, disc, dinvc, b, g, be, Wn)


def _ep_body(part_ref, hw_ref, h_ref, disc_ref, dinvc_ref, b_ref, g_ref, be_ref, out_ref, *, relu):
    o = (part_ref[0] + part_ref[1]) * disc_ref[...]
    o = o + hw_ref[...] * dinvc_ref[...] + b_ref[0][None, :]
    mu = jnp.mean(o, axis=-1, keepdims=True)
    v = jnp.mean((o - mu) ** 2, axis=-1, keepdims=True)
    o = (o - mu) * lax.rsqrt(v + 1e-5) * g_ref[0][None, :] + be_ref[0][None, :]
    if relu:
        o = jnp.maximum(o, 0.0)
    out_ref[...] = o + h_ref[...]


def _epilogue(part, hw, h, disc, dinvc, b, g, be, relu):
    grid = (N_PAD // RB,)
    return pl.pallas_call(
        functools.partial(_ep_body, relu=relu),
        grid=grid,
        in_specs=[
            pl.BlockSpec((NC, RB, D), lambda i: (0, i, 0)),
            pl.BlockSpec((RB, D), lambda i: (i, 0)),
            pl.BlockSpec((RB, D), lambda i: (i, 0)),
            pl.BlockSpec((RB, 1), lambda i: (i, 0)),
            pl.BlockSpec((RB, 1), lambda i: (i, 0)),
            pl.BlockSpec((1, D), lambda i: (0, 0)),
            pl.BlockSpec((1, D), lambda i: (0, 0)),
            pl.BlockSpec((1, D), lambda i: (0, 0)),
        ],
        out_specs=pl.BlockSpec((RB, D), lambda i: (i, 0)),
        out_shape=jax.ShapeDtypeStruct((N_PAD, D), jnp.float32),
    )(part, hw, h, disc, dinvc, b, g, be)


def kernel(x, edge_index, edge_weight, W1, b1, W2, b2, W3, b3, g1, be1, g2, be2, g3, be3):
    src = edge_index[0].astype(jnp.int32)
    dst = edge_index[1].astype(jnp.int32)
    ew = edge_weight.astype(jnp.float32)
    pad = E_PAD - E
    src3 = jnp.pad(src, (0, pad)).reshape(T_CHUNKS, CHUNK)
    dst3 = jnp.pad(dst, (0, pad)).reshape(T_CHUNKS, CHUNK)
    ew3 = jnp.pad(ew, (0, pad)).reshape(T_CHUNKS, CHUNK)

    degp = _deg_partials(dst3, ew3)
    dis2, dinv2 = _finalize(degp)
    disc = dis2.reshape(N_PAD, 1)
    dinvc = dinv2.reshape(N_PAD, 1)

    h = jnp.pad(x, ((0, N_PAD - N), (0, 0)))
    hw, hws = _matmul_scaled(h, W1, disc)
    part = _message_pass(hws, src3, dst3, ew3)
    h1, hw2, hws2 = _ep_matmul(part, hw, h, disc, dinvc,
                               b1.reshape(1, D), g1.reshape(1, D),
                               be1.reshape(1, D), W2)
    part = _message_pass(hws2, src3, dst3, ew3)
    h2, hw3, hws3 = _ep_matmul(part, hw2, h1, disc, dinvc,
                               b2.reshape(1, D), g2.reshape(1, D),
                               be2.reshape(1, D), W3)
    part = _message_pass(hws3, src3, dst3, ew3)
    h3 = _epilogue(part, hw3, h2, disc, dinvc,
                   b3.reshape(1, D), g3.reshape(1, D), be3.reshape(1, D), False)
    return h3[:N]
